# Initial kernel scaffold; baseline (speedup 1.0000x reference)
import jax
import jax.numpy as jnp
from jax.experimental import pallas as pl


def kernel(features, edge_index, W1, b1, W2, b2):
    raise NotImplementedError("write your pallas kernel here")


# speedup vs baseline: 9.0617x; 9.0617x over previous
import dataclasses
import functools

import jax
import jax.numpy as jnp
from jax import lax
from jax.experimental import pallas as pl
from jax.experimental.pallas import tpu as pltpu
from jax.experimental.pallas import tpu_sc as plsc

N_NODES = 10000
N_EDGES = 160000
F_IN = 256
HID = 16

NC, NS, LANES = 2, 16, 16
NW = NC * NS
WIN = 128
N_PAD = 10240
DUMP = N_NODES
ROWS_PER_TILE = N_PAD // NS
ZCHUNK = 128
NZ = ROWS_PER_TILE // ZCHUNK
NWINP = ((N_EDGES + WIN - 1) // WIN + NW - 1) // NW * NW
WPW = NWINP // NW
E_PAD = NWINP * WIN

_mesh = plsc.VectorSubcoreMesh(core_axis_name="c", subcore_axis_name="s")

_sc_params = pltpu.CompilerParams(
    needs_layout_passes=False, use_tc_tiling_on_sc=False)


def _deg_body(srcp, dstp, out, hs, hd, si, di):
    cid = lax.axis_index("c")
    sid = lax.axis_index("s")
    wid = cid * NS + sid
    zeros = jnp.zeros((LANES,), jnp.float32)

    @pl.loop(0, N_PAD // LANES)
    def _(i):
        hs[pl.ds(i * LANES, LANES)] = zeros
        hd[pl.ds(i * LANES, LANES)] = zeros

    pltpu.sync_copy(srcp.at[pl.ds(wid * WPW, WPW)], si)
    pltpu.sync_copy(dstp.at[pl.ds(wid * WPW, WPW)], di)
    ones = jnp.ones((LANES,), jnp.float32)

    @pl.loop(0, WPW)
    def _(j):
        @pl.loop(0, WIN // LANES)
        def _(v):
            plsc.addupdate_scatter(hs, [si[j, pl.ds(v * LANES, LANES)]], ones)
            plsc.addupdate_scatter(hd, [di[j, pl.ds(v * LANES, LANES)]], ones)

    pltpu.sync_copy(hs, out.at[wid, 0])
    pltpu.sync_copy(hd, out.at[wid, 1])


def _sc_degrees(srcp, dstp):
    k = pl.kernel(
        _deg_body,
        out_type=jax.ShapeDtypeStruct((NW, 2, N_PAD), jnp.float32),
        mesh=_mesh,
        scratch_types=[
            pltpu.VMEM((N_PAD,), jnp.float32),
            pltpu.VMEM((N_PAD,), jnp.float32),
            pltpu.VMEM((WPW, WIN), jnp.int32),
            pltpu.VMEM((WPW, WIN), jnp.int32),
        ],
        compiler_params=_sc_params,
    )
    return k(srcp, dstp)


def _agg_body(h, srcp, dstp, out, si, di, rows, zb, acc, sem):
    cid = lax.axis_index("c")
    sid = lax.axis_index("s")
    wid = cid * NS + sid
    zeros = jnp.zeros((LANES,), jnp.float32)

    @pl.loop(0, ZCHUNK)
    def _(i):
        zb[i, :] = zeros

    @pl.loop(0, NZ)
    def _(kk):
        pltpu.sync_copy(zb, acc.at[pl.ds(sid * ROWS_PER_TILE + kk * ZCHUNK, ZCHUNK)])

    pltpu.sync_copy(srcp.at[pl.ds(wid * WPW, WPW)], si)
    pltpu.sync_copy(dstp.at[pl.ds(wid * WPW, WPW)], di)
    plsc.subcore_barrier()

    @pl.loop(0, WPW)
    def _(j):
        pltpu.async_copy(h.at[si.at[j]], rows, sem).wait()
        pltpu.sync_copy(rows, acc.at[di.at[j]], add=True)

    plsc.subcore_barrier()
    pltpu.sync_copy(acc.at[pl.ds(sid * ROWS_PER_TILE, ROWS_PER_TILE)],
                    out.at[cid, pl.ds(sid * ROWS_PER_TILE, ROWS_PER_TILE)])


def _sc_agg(h, srcp, dstp):
    k = pl.kernel(
        _agg_body,
        out_type=jax.ShapeDtypeStruct((NC, N_PAD, HID), jnp.float32),
        mesh=_mesh,
        scratch_types=[
            pltpu.VMEM((WPW, WIN), jnp.int32),
            pltpu.VMEM((WPW, WIN), jnp.int32),
            pltpu.VMEM((WIN, HID), jnp.float32),
            pltpu.VMEM((ZCHUNK, HID), jnp.float32),
            pltpu.VMEM_SHARED((N_PAD, HID), jnp.float32),
            pltpu.SemaphoreType.DMA,
        ],
        compiler_params=_sc_params,
    )
    return k(h, srcp, dstp)


_BM = 1024


def _mm1_body(x_ref, w_ref, degp_ref, h_ref, norm_ref):
    deg = jnp.maximum(jnp.sum(degp_ref[...], axis=0), 1.0)
    norm = lax.rsqrt(deg)
    norm_ref[...] = norm
    mm = jnp.dot(x_ref[...], w_ref[...], preferred_element_type=jnp.float32)
    h_ref[...] = mm * norm[0][:, None]


def _tc_mm1(features, W1, degp):
    grid = N_PAD // _BM
    return pl.pallas_call(
        _mm1_body,
        grid=(grid,),
        in_specs=[
            pl.BlockSpec((_BM, F_IN), lambda i: (i, 0)),
            pl.BlockSpec((F_IN, HID), lambda i: (0, 0)),
            pl.BlockSpec((NW, 2, _BM), lambda i: (0, 0, i)),
        ],
        out_specs=[
            pl.BlockSpec((_BM, HID), lambda i: (i, 0)),
            pl.BlockSpec((2, _BM), lambda i: (0, i)),
        ],
        out_shape=[
            jax.ShapeDtypeStruct((N_PAD, HID), jnp.float32),
            jax.ShapeDtypeStruct((2, N_PAD), jnp.float32),
        ],
    )(features, W1, degp)


def _mid_body(p_ref, norm_ref, b1_ref, w2_ref, h2_ref):
    agg = p_ref[0] + p_ref[1]
    t = agg * norm_ref[1][:, None] + b1_ref[...]
    t = jnp.maximum(t, 0.0)
    mm = jnp.dot(t, w2_ref[...], preferred_element_type=jnp.float32)
    h2_ref[...] = mm * norm_ref[0][:, None]


def _tc_mid(p1, norms, b1, W2):
    grid = N_PAD // _BM
    return pl.pallas_call(
        _mid_body,
        grid=(grid,),
        in_specs=[
            pl.BlockSpec((NC, _BM, HID), lambda i: (0, i, 0)),
            pl.BlockSpec((2, _BM), lambda i: (0, i)),
            pl.BlockSpec((1, HID), lambda i: (0, 0)),
            pl.BlockSpec((HID, HID), lambda i: (0, 0)),
        ],
        out_specs=pl.BlockSpec((_BM, HID), lambda i: (i, 0)),
        out_shape=jax.ShapeDtypeStruct((N_PAD, HID), jnp.float32),
    )(p1, norms, b1, W2)


def _final_body(p_ref, norm_ref, b2_ref, out_ref):
    agg = p_ref[0] + p_ref[1]
    out_ref[...] = agg * norm_ref[1][:, None] + b2_ref[...]


def _tc_final(p2, norms, b2):
    grid = pl.cdiv(N_NODES, _BM)
    return pl.pallas_call(
        _final_body,
        grid=(grid,),
        in_specs=[
            pl.BlockSpec((NC, _BM, HID), lambda i: (0, i, 0)),
            pl.BlockSpec((2, _BM), lambda i: (0, i)),
            pl.BlockSpec((1, HID), lambda i: (0, 0)),
        ],
        out_specs=pl.BlockSpec((_BM, HID), lambda i: (i, 0)),
        out_shape=jax.ShapeDtypeStruct((N_NODES, HID), jnp.float32),
    )(p2, norms, b2)


def kernel(features, edge_index, W1, b1, W2, b2):
    src = edge_index[0].astype(jnp.int32)
    dst = edge_index[1].astype(jnp.int32)
    pad = E_PAD - N_EDGES
    fill = jnp.full((pad,), DUMP, jnp.int32)
    srcp = jnp.concatenate([src, fill]).reshape(NWINP, WIN)
    dstp = jnp.concatenate([dst, fill]).reshape(NWINP, WIN)

    degp = _sc_degrees(srcp, dstp)
    h1, norms = _tc_mm1(features, W1, degp)
    p1 = _sc_agg(h1, srcp, dstp)
    h2 = _tc_mid(p1, norms, b1.reshape(1, HID), W2)
    p2 = _sc_agg(h2, srcp, dstp)
    return _tc_final(p2, norms, b2.reshape(1, HID))


# speedup vs baseline: 14.6347x; 1.6150x over previous
import dataclasses
import functools

import jax
import jax.numpy as jnp
from jax import lax
from jax.experimental import pallas as pl
from jax.experimental.pallas import tpu as pltpu
from jax.experimental.pallas import tpu_sc as plsc

N_NODES = 10000
N_EDGES = 160000
F_IN = 256
HID = 16

NC, NS, LANES = 2, 16, 16
NW = NC * NS
WIN = 128
N_PAD = 10240
DUMP = N_NODES
ROWS_PER_TILE = N_PAD // NS
ZCHUNK = 128
NZ = ROWS_PER_TILE // ZCHUNK
NWINP = ((N_EDGES + WIN - 1) // WIN + NW - 1) // NW * NW
WPW = NWINP // NW
E_PAD = NWINP * WIN
EPW = E_PAD // NW

_mesh = plsc.VectorSubcoreMesh(core_axis_name="c", subcore_axis_name="s")

_sc_params = pltpu.CompilerParams(
    needs_layout_passes=False, use_tc_tiling_on_sc=False)


def _deg_body(srcp, dstp, out, hs, hd, si, di):
    cid = lax.axis_index("c")
    sid = lax.axis_index("s")
    wid = cid * NS + sid
    zeros = jnp.zeros((LANES,), jnp.float32)

    @pl.loop(0, N_PAD // LANES)
    def _(i):
        hs[pl.ds(i * LANES, LANES)] = zeros
        hd[pl.ds(i * LANES, LANES)] = zeros

    pltpu.sync_copy(srcp.at[pl.ds(wid * WPW, WPW)], si)
    pltpu.sync_copy(dstp.at[pl.ds(wid * WPW, WPW)], di)
    ones = jnp.ones((LANES,), jnp.float32)

    @pl.loop(0, WPW)
    def _(j):
        @pl.loop(0, WIN // LANES)
        def _(v):
            plsc.addupdate_scatter(hs, [si[j, pl.ds(v * LANES, LANES)]], ones)
            plsc.addupdate_scatter(hd, [di[j, pl.ds(v * LANES, LANES)]], ones)

    pltpu.sync_copy(hs, out.at[wid, 0])
    pltpu.sync_copy(hd, out.at[wid, 1])


def _sc_degrees(srcp, dstp):
    k = pl.kernel(
        _deg_body,
        out_type=jax.ShapeDtypeStruct((NW, 2, N_PAD), jnp.float32),
        mesh=_mesh,
        scratch_types=[
            pltpu.VMEM((N_PAD,), jnp.float32),
            pltpu.VMEM((N_PAD,), jnp.float32),
            pltpu.VMEM((WPW, WIN), jnp.int32),
            pltpu.VMEM((WPW, WIN), jnp.int32),
        ],
        compiler_params=_sc_params,
    )
    return k(srcp, dstp)


def _agg_body(h, srcf, dstf, out, si, di, rows, zb, acc, sem):
    cid = lax.axis_index("c")
    sid = lax.axis_index("s")
    wid = cid * NS + sid
    zeros = jnp.zeros((LANES,), jnp.float32)

    @pl.loop(0, ZCHUNK)
    def _(i):
        zb[i, :] = zeros

    @pl.loop(0, NZ)
    def _(kk):
        pltpu.sync_copy(zb, acc.at[pl.ds(sid * ROWS_PER_TILE + kk * ZCHUNK, ZCHUNK)])

    pltpu.sync_copy(srcf.at[pl.ds(wid * EPW, EPW)], si)
    pltpu.sync_copy(dstf.at[pl.ds(wid * EPW, EPW)], di)
    plsc.subcore_barrier()

    pltpu.async_copy(h.at[si], rows, sem).wait()
    pltpu.sync_copy(rows, acc.at[di], add=True)

    plsc.subcore_barrier()
    pltpu.sync_copy(acc.at[pl.ds(sid * ROWS_PER_TILE, ROWS_PER_TILE)],
                    out.at[cid, pl.ds(sid * ROWS_PER_TILE, ROWS_PER_TILE)])


def _sc_agg(h, srcf, dstf):
    k = pl.kernel(
        _agg_body,
        out_type=jax.ShapeDtypeStruct((NC, N_PAD, HID), jnp.float32),
        mesh=_mesh,
        scratch_types=[
            pltpu.VMEM((EPW,), jnp.int32),
            pltpu.VMEM((EPW,), jnp.int32),
            pltpu.VMEM((EPW, HID), jnp.float32),
            pltpu.VMEM((ZCHUNK, HID), jnp.float32),
            pltpu.VMEM_SHARED((N_PAD, HID), jnp.float32),
            pltpu.SemaphoreType.DMA,
        ],
        compiler_params=_sc_params,
    )
    return k(h, srcf, dstf)


_BM = 1024


def _mm1_body(x_ref, w_ref, degp_ref, h_ref, norm_ref):
    deg = jnp.maximum(jnp.sum(degp_ref[...], axis=0), 1.0)
    norm = lax.rsqrt(deg)
    norm_ref[...] = norm
    mm = jnp.dot(x_ref[...], w_ref[...], preferred_element_type=jnp.float32)
    h_ref[...] = mm * norm[0][:, None]


def _tc_mm1(features, W1, degp):
    grid = N_PAD // _BM
    return pl.pallas_call(
        _mm1_body,
        grid=(grid,),
        in_specs=[
            pl.BlockSpec((_BM, F_IN), lambda i: (i, 0)),
            pl.BlockSpec((F_IN, HID), lambda i: (0, 0)),
            pl.BlockSpec((NW, 2, _BM), lambda i: (0, 0, i)),
        ],
        out_specs=[
            pl.BlockSpec((_BM, HID), lambda i: (i, 0)),
            pl.BlockSpec((2, _BM), lambda i: (0, i)),
        ],
        out_shape=[
            jax.ShapeDtypeStruct((N_PAD, HID), jnp.float32),
            jax.ShapeDtypeStruct((2, N_PAD), jnp.float32),
        ],
    )(features, W1, degp)


def _mid_body(p_ref, norm_ref, b1_ref, w2_ref, h2_ref):
    agg = p_ref[0] + p_ref[1]
    t = agg * norm_ref[1][:, None] + b1_ref[...]
    t = jnp.maximum(t, 0.0)
    mm = jnp.dot(t, w2_ref[...], preferred_element_type=jnp.float32)
    h2_ref[...] = mm * norm_ref[0][:, None]


def _tc_mid(p1, norms, b1, W2):
    grid = N_PAD // _BM
    return pl.pallas_call(
        _mid_body,
        grid=(grid,),
        in_specs=[
            pl.BlockSpec((NC, _BM, HID), lambda i: (0, i, 0)),
            pl.BlockSpec((2, _BM), lambda i: (0, i)),
            pl.BlockSpec((1, HID), lambda i: (0, 0)),
            pl.BlockSpec((HID, HID), lambda i: (0, 0)),
        ],
        out_specs=pl.BlockSpec((_BM, HID), lambda i: (i, 0)),
        out_shape=jax.ShapeDtypeStruct((N_PAD, HID), jnp.float32),
    )(p1, norms, b1, W2)


def _final_body(p_ref, norm_ref, b2_ref, out_ref):
    agg = p_ref[0] + p_ref[1]
    out_ref[...] = agg * norm_ref[1][:, None] + b2_ref[...]


def _tc_final(p2, norms, b2):
    grid = pl.cdiv(N_NODES, _BM)
    return pl.pallas_call(
        _final_body,
        grid=(grid,),
        in_specs=[
            pl.BlockSpec((NC, _BM, HID), lambda i: (0, i, 0)),
            pl.BlockSpec((2, _BM), lambda i: (0, i)),
            pl.BlockSpec((1, HID), lambda i: (0, 0)),
        ],
        out_specs=pl.BlockSpec((_BM, HID), lambda i: (i, 0)),
        out_shape=jax.ShapeDtypeStruct((N_NODES, HID), jnp.float32),
    )(p2, norms, b2)


def kernel(features, edge_index, W1, b1, W2, b2):
    src = edge_index[0].astype(jnp.int32)
    dst = edge_index[1].astype(jnp.int32)
    pad = E_PAD - N_EDGES
    fill = DUMP + jnp.arange(pad, dtype=jnp.int32) % (N_PAD - N_NODES)
    srcf = jnp.concatenate([src, fill])
    dstf = jnp.concatenate([dst, fill])
    srcp = srcf.reshape(NWINP, WIN)
    dstp = dstf.reshape(NWINP, WIN)

    degp = _sc_degrees(srcp, dstp)
    h1, norms = _tc_mm1(features, W1, degp)
    p1 = _sc_agg(h1, srcf, dstf)
    h2 = _tc_mid(p1, norms, b1.reshape(1, HID), W2)
    p2 = _sc_agg(h2, srcf, dstf)
    return _tc_final(p2, norms, b2.reshape(1, HID))


# speedup vs baseline: 16.0303x; 1.0954x over previous
import dataclasses
import functools

import jax
import jax.numpy as jnp
from jax import lax
from jax.experimental import pallas as pl
from jax.experimental.pallas import tpu as pltpu
from jax.experimental.pallas import tpu_sc as plsc

N_NODES = 10000
N_EDGES = 160000
F_IN = 256
HID = 16

NC, NS, LANES = 2, 16, 16
NW = NC * NS
WIN = 128
N_PAD = 10240
DUMP = N_NODES
ROWS_PER_TILE = N_PAD // NS
ZCHUNK = 128
NZ = ROWS_PER_TILE // ZCHUNK
EPT = N_EDGES // NW
VFULL = EPT // LANES
TAIL = EPT - VFULL * LANES

_mesh = plsc.VectorSubcoreMesh(core_axis_name="c", subcore_axis_name="s")

_sc_params = pltpu.CompilerParams(
    needs_layout_passes=False, use_tc_tiling_on_sc=False)


def _deg_body(ei, out, hs, hd, si, di):
    cid = lax.axis_index("c")
    sid = lax.axis_index("s")
    wid = cid * NS + sid
    zeros = jnp.zeros((LANES,), jnp.float32)

    @pl.loop(0, N_PAD // LANES)
    def _(i):
        hs[pl.ds(i * LANES, LANES)] = zeros
        hd[pl.ds(i * LANES, LANES)] = zeros

    pltpu.sync_copy(ei.at[0, pl.ds(wid * EPT, EPT)], si.at[pl.ds(0, EPT)])
    pltpu.sync_copy(ei.at[1, pl.ds(wid * EPT, EPT)], di.at[pl.ds(0, EPT)])
    ones = jnp.ones((LANES,), jnp.float32)

    @pl.loop(0, VFULL)
    def _(v):
        plsc.addupdate_scatter(hs, [si[pl.ds(v * LANES, LANES)]], ones)
        plsc.addupdate_scatter(hd, [di[pl.ds(v * LANES, LANES)]], ones)

    tmask = lax.iota(jnp.int32, LANES) < TAIL
    plsc.addupdate_scatter(hs, [si[pl.ds(VFULL * LANES, LANES)]], ones, mask=tmask)
    plsc.addupdate_scatter(hd, [di[pl.ds(VFULL * LANES, LANES)]], ones, mask=tmask)

    pltpu.sync_copy(hs, out.at[0, wid])
    pltpu.sync_copy(hd, out.at[1, wid])


def _sc_degrees(ei):
    k = pl.kernel(
        _deg_body,
        out_type=jax.ShapeDtypeStruct((2, NW, N_PAD), jnp.float32),
        mesh=_mesh,
        scratch_types=[
            pltpu.VMEM((N_PAD,), jnp.float32),
            pltpu.VMEM((N_PAD,), jnp.float32),
            pltpu.VMEM((VFULL * LANES + LANES,), jnp.int32),
            pltpu.VMEM((VFULL * LANES + LANES,), jnp.int32),
        ],
        compiler_params=_sc_params,
    )
    return k(ei)


def _agg_body(h, ei, out, si, di, rows, zb, acc, sem):
    cid = lax.axis_index("c")
    sid = lax.axis_index("s")
    wid = cid * NS + sid
    zeros = jnp.zeros((LANES,), jnp.float32)

    @pl.loop(0, ZCHUNK)
    def _(i):
        zb[i, :] = zeros

    @pl.loop(0, NZ)
    def _(kk):
        pltpu.sync_copy(zb, acc.at[pl.ds(sid * ROWS_PER_TILE + kk * ZCHUNK, ZCHUNK)])

    pltpu.sync_copy(ei.at[0, pl.ds(wid * EPT, EPT)], si)
    pltpu.sync_copy(ei.at[1, pl.ds(wid * EPT, EPT)], di)
    plsc.subcore_barrier()

    pltpu.async_copy(h.at[si], rows, sem).wait()
    pltpu.sync_copy(rows, acc.at[di], add=True)

    plsc.subcore_barrier()
    pltpu.sync_copy(acc.at[pl.ds(sid * ROWS_PER_TILE, ROWS_PER_TILE)],
                    out.at[cid, pl.ds(sid * ROWS_PER_TILE, ROWS_PER_TILE)])


def _sc_agg(h, ei):
    k = pl.kernel(
        _agg_body,
        out_type=jax.ShapeDtypeStruct((NC, N_PAD, HID), jnp.float32),
        mesh=_mesh,
        scratch_types=[
            pltpu.VMEM((EPT,), jnp.int32),
            pltpu.VMEM((EPT,), jnp.int32),
            pltpu.VMEM((EPT, HID), jnp.float32),
            pltpu.VMEM((ZCHUNK, HID), jnp.float32),
            pltpu.VMEM_SHARED((N_PAD, HID), jnp.float32),
            pltpu.SemaphoreType.DMA,
        ],
        compiler_params=_sc_params,
    )
    return k(h, ei)


_BM = 1024


def _mm1_body(x_ref, w_ref, degp_ref, h_ref, norm_ref):
    deg = jnp.maximum(jnp.sum(degp_ref[...], axis=1), 1.0)
    norm = lax.rsqrt(deg)
    norm_ref[...] = norm
    mm = jnp.dot(x_ref[...], w_ref[...], preferred_element_type=jnp.float32)
    h_ref[...] = mm * norm[0][:, None]


def _tc_mm1(features, W1, degp):
    grid = N_PAD // _BM
    return pl.pallas_call(
        _mm1_body,
        grid=(grid,),
        in_specs=[
            pl.BlockSpec((_BM, F_IN), lambda i: (i, 0)),
            pl.BlockSpec((F_IN, HID), lambda i: (0, 0)),
            pl.BlockSpec((2, NW, _BM), lambda i: (0, 0, i)),
        ],
        out_specs=[
            pl.BlockSpec((_BM, HID), lambda i: (i, 0)),
            pl.BlockSpec((2, _BM), lambda i: (0, i)),
        ],
        out_shape=[
            jax.ShapeDtypeStruct((N_PAD, HID), jnp.float32),
            jax.ShapeDtypeStruct((2, N_PAD), jnp.float32),
        ],
    )(features, W1, degp)


def _mid_body(p_ref, norm_ref, b1_ref, w2_ref, h2_ref):
    agg = p_ref[0] + p_ref[1]
    t = agg * norm_ref[1][:, None] + b1_ref[...]
    t = jnp.maximum(t, 0.0)
    mm = jnp.dot(t, w2_ref[...], preferred_element_type=jnp.float32)
    h2_ref[...] = mm * norm_ref[0][:, None]


def _tc_mid(p1, norms, b1, W2):
    grid = N_PAD // _BM
    return pl.pallas_call(
        _mid_body,
        grid=(grid,),
        in_specs=[
            pl.BlockSpec((NC, _BM, HID), lambda i: (0, i, 0)),
            pl.BlockSpec((2, _BM), lambda i: (0, i)),
            pl.BlockSpec((1, HID), lambda i: (0, 0)),
            pl.BlockSpec((HID, HID), lambda i: (0, 0)),
        ],
        out_specs=pl.BlockSpec((_BM, HID), lambda i: (i, 0)),
        out_shape=jax.ShapeDtypeStruct((N_PAD, HID), jnp.float32),
    )(p1, norms, b1, W2)


def _final_body(p_ref, norm_ref, b2_ref, out_ref):
    agg = p_ref[0] + p_ref[1]
    out_ref[...] = agg * norm_ref[1][:, None] + b2_ref[...]


def _tc_final(p2, norms, b2):
    grid = pl.cdiv(N_NODES, _BM)
    return pl.pallas_call(
        _final_body,
        grid=(grid,),
        in_specs=[
            pl.BlockSpec((NC, _BM, HID), lambda i: (0, i, 0)),
            pl.BlockSpec((2, _BM), lambda i: (0, i)),
            pl.BlockSpec((1, HID), lambda i: (0, 0)),
        ],
        out_specs=pl.BlockSpec((_BM, HID), lambda i: (i, 0)),
        out_shape=jax.ShapeDtypeStruct((N_NODES, HID), jnp.float32),
    )(p2, norms, b2)


def kernel(features, edge_index, W1, b1, W2, b2):
    ei = edge_index.astype(jnp.int32)

    degp = _sc_degrees(ei)
    h1, norms = _tc_mm1(features, W1, degp)
    p1 = _sc_agg(h1, ei)
    h2 = _tc_mid(p1, norms, b1.reshape(1, HID), W2)
    p2 = _sc_agg(h2, ei)
    return _tc_final(p2, norms, b2.reshape(1, HID))


# speedup vs baseline: 17.1779x; 1.0716x over previous
import dataclasses
import functools

import jax
import jax.numpy as jnp
from jax import lax
from jax.experimental import pallas as pl
from jax.experimental.pallas import tpu as pltpu
from jax.experimental.pallas import tpu_sc as plsc

N_NODES = 10000
N_EDGES = 160000
F_IN = 256
HID = 16

NC, NS, LANES = 2, 16, 16
NW = NC * NS
WIN = 128
N_PAD = 10240
DUMP = N_NODES
ROWS_PER_TILE = N_PAD // NS
ZCHUNK = 128
NZ = ROWS_PER_TILE // ZCHUNK
EPT = N_EDGES // NW
VFULL = EPT // LANES
TAIL = EPT - VFULL * LANES

_mesh = plsc.VectorSubcoreMesh(core_axis_name="c", subcore_axis_name="s")

_sc_params = pltpu.CompilerParams(
    needs_layout_passes=False, use_tc_tiling_on_sc=False)


def _deg_body(ei, out, hs, hd, si, di):
    cid = lax.axis_index("c")
    sid = lax.axis_index("s")
    wid = cid * NS + sid
    zeros = jnp.zeros((LANES,), jnp.float32)

    @pl.loop(0, N_PAD // LANES)
    def _(i):
        hs[pl.ds(i * LANES, LANES)] = zeros
        hd[pl.ds(i * LANES, LANES)] = zeros

    pltpu.sync_copy(ei.at[0, pl.ds(wid * EPT, EPT)], si.at[pl.ds(0, EPT)])
    pltpu.sync_copy(ei.at[1, pl.ds(wid * EPT, EPT)], di.at[pl.ds(0, EPT)])
    ones = jnp.ones((LANES,), jnp.float32)

    @pl.loop(0, VFULL)
    def _(v):
        plsc.addupdate_scatter(hs, [si[pl.ds(v * LANES, LANES)]], ones)
        plsc.addupdate_scatter(hd, [di[pl.ds(v * LANES, LANES)]], ones)

    tmask = lax.iota(jnp.int32, LANES) < TAIL
    plsc.addupdate_scatter(hs, [si[pl.ds(VFULL * LANES, LANES)]], ones, mask=tmask)
    plsc.addupdate_scatter(hd, [di[pl.ds(VFULL * LANES, LANES)]], ones, mask=tmask)

    pltpu.sync_copy(hs, out.at[0, wid])
    pltpu.sync_copy(hd, out.at[1, wid])


def _sc_degrees(ei):
    k = pl.kernel(
        _deg_body,
        out_type=jax.ShapeDtypeStruct((2, NW, N_PAD), jnp.float32),
        mesh=_mesh,
        scratch_types=[
            pltpu.VMEM((N_PAD,), jnp.float32),
            pltpu.VMEM((N_PAD,), jnp.float32),
            pltpu.VMEM((VFULL * LANES + LANES,), jnp.int32),
            pltpu.VMEM((VFULL * LANES + LANES,), jnp.int32),
        ],
        compiler_params=_sc_params,
    )
    return k(ei)


def _agg_body(h, ei, out, si, di, rows, zb, acc, sem):
    cid = lax.axis_index("c")
    sid = lax.axis_index("s")
    wid = cid * NS + sid
    zeros = jnp.zeros((LANES,), jnp.float32)

    @pl.loop(0, ZCHUNK)
    def _(i):
        zb[i, :] = zeros

    @pl.loop(0, NZ)
    def _(kk):
        pltpu.sync_copy(zb, acc.at[pl.ds(sid * ROWS_PER_TILE + kk * ZCHUNK, ZCHUNK)])

    pltpu.sync_copy(ei.at[0, pl.ds(wid * EPT, EPT)], si)
    pltpu.sync_copy(ei.at[1, pl.ds(wid * EPT, EPT)], di)
    plsc.subcore_barrier()

    pltpu.async_copy(h.at[si], rows, sem).wait()
    pltpu.sync_copy(rows, acc.at[di], add=True)

    plsc.subcore_barrier()
    pltpu.sync_copy(acc.at[pl.ds(sid * ROWS_PER_TILE, ROWS_PER_TILE)],
                    out.at[cid, pl.ds(sid * ROWS_PER_TILE, ROWS_PER_TILE)])


def _sc_agg(h, ei):
    k = pl.kernel(
        _agg_body,
        out_type=jax.ShapeDtypeStruct((NC, N_PAD, HID), jnp.float32),
        mesh=_mesh,
        scratch_types=[
            pltpu.VMEM((EPT,), jnp.int32),
            pltpu.VMEM((EPT,), jnp.int32),
            pltpu.VMEM((EPT, HID), jnp.float32),
            pltpu.VMEM((ZCHUNK, HID), jnp.float32),
            pltpu.VMEM_SHARED((N_PAD, HID), jnp.float32),
            pltpu.SemaphoreType.DMA,
        ],
        compiler_params=_sc_params,
    )
    return k(h, ei)


_BM = 1024
_BMP = _BM // (128 // HID)
PROWS = N_PAD // (128 // HID)


def _pack_consts():
    r8 = 128 // HID
    m_row = lax.broadcasted_iota(jnp.int32, (_BM, 128), 0)
    c_col = lax.broadcasted_iota(jnp.int32, (_BM, 128), 1)
    mask = (m_row % r8 == c_col // HID).astype(jnp.float32)
    j_row = lax.broadcasted_iota(jnp.int32, (HID, 128), 0)
    c2 = lax.broadcasted_iota(jnp.int32, (HID, 128), 1)
    e = (j_row == c2 % HID).astype(jnp.float32)
    r_row = lax.broadcasted_iota(jnp.int32, (_BMP, _BM), 0)
    m_col = lax.broadcasted_iota(jnp.int32, (_BMP, _BM), 1)
    sel = (m_col // r8 == r_row).astype(jnp.float32)
    return sel, e, mask


_HI = lax.Precision.HIGHEST


def _pack(x, sel, e, mask):
    ext = jnp.dot(x, e, preferred_element_type=jnp.float32, precision=_HI) * mask
    return jnp.dot(sel, ext, preferred_element_type=jnp.float32, precision=_HI)


def _mm1_body(x_ref, w_ref, degp_ref, hp_ref, nrmd_ref, nrms_ref):
    deg = jnp.maximum(jnp.sum(degp_ref[...], axis=1), 1.0)
    norm = lax.rsqrt(deg)
    sel, e, mask = _pack_consts()
    nrms_col = norm[0][:, None]
    nrmd_col = norm[1][:, None]
    ones_ext = jnp.ones((_BM, 128), jnp.float32)
    nrms_ref[...] = jnp.dot(sel, nrms_col * ones_ext * mask,
                            preferred_element_type=jnp.float32, precision=_HI)
    nrmd_ref[...] = jnp.dot(sel, nrmd_col * ones_ext * mask,
                            preferred_element_type=jnp.float32, precision=_HI)
    mm = jnp.dot(x_ref[...], w_ref[...], preferred_element_type=jnp.float32)
    hp_ref[...] = _pack(mm * nrms_col, sel, e, mask)


def _tc_mm1(features, W1, degp):
    grid = N_PAD // _BM
    return pl.pallas_call(
        _mm1_body,
        grid=(grid,),
        in_specs=[
            pl.BlockSpec((_BM, F_IN), lambda i: (i, 0)),
            pl.BlockSpec((F_IN, HID), lambda i: (0, 0)),
            pl.BlockSpec((2, NW, _BM), lambda i: (0, 0, i)),
        ],
        out_specs=[
            pl.BlockSpec((_BMP, 128), lambda i: (i, 0)),
            pl.BlockSpec((_BMP, 128), lambda i: (i, 0)),
            pl.BlockSpec((_BMP, 128), lambda i: (i, 0)),
        ],
        out_shape=[
            jax.ShapeDtypeStruct((PROWS, 128), jnp.float32),
            jax.ShapeDtypeStruct((PROWS, 128), jnp.float32),
            jax.ShapeDtypeStruct((PROWS, 128), jnp.float32),
        ],
    )(features, W1, degp)


def _mid_body(p_ref, nrmd_ref, nrms_ref, b1_ref, w2bd_ref, h2p_ref):
    agg = p_ref[0] + p_ref[1]
    t = jnp.maximum(agg * nrmd_ref[...] + b1_ref[...], 0.0)
    mm = jnp.dot(t, w2bd_ref[...], preferred_element_type=jnp.float32)
    h2p_ref[...] = mm * nrms_ref[...]


def _tc_mid(p1, nrmd_p, nrms_p, b1t, W2bd):
    grid = PROWS // _BMP
    return pl.pallas_call(
        _mid_body,
        grid=(grid,),
        in_specs=[
            pl.BlockSpec((NC, _BMP, 128), lambda i: (0, i, 0)),
            pl.BlockSpec((_BMP, 128), lambda i: (i, 0)),
            pl.BlockSpec((_BMP, 128), lambda i: (i, 0)),
            pl.BlockSpec((1, 128), lambda i: (0, 0)),
            pl.BlockSpec((128, 128), lambda i: (0, 0)),
        ],
        out_specs=pl.BlockSpec((_BMP, 128), lambda i: (i, 0)),
        out_shape=jax.ShapeDtypeStruct((PROWS, 128), jnp.float32),
    )(p1, nrmd_p, nrms_p, b1t, W2bd)


def _final_body(p_ref, nrmd_ref, b2_ref, out_ref):
    agg = p_ref[0] + p_ref[1]
    out_ref[...] = agg * nrmd_ref[...] + b2_ref[...]


def _tc_final(p2, nrmd_p, b2t):
    grid = PROWS // _BMP
    return pl.pallas_call(
        _final_body,
        grid=(grid,),
        in_specs=[
            pl.BlockSpec((NC, _BMP, 128), lambda i: (0, i, 0)),
            pl.BlockSpec((_BMP, 128), lambda i: (i, 0)),
            pl.BlockSpec((1, 128), lambda i: (0, 0)),
        ],
        out_specs=pl.BlockSpec((_BMP, 128), lambda i: (i, 0)),
        out_shape=jax.ShapeDtypeStruct((PROWS, 128), jnp.float32),
    )(p2, nrmd_p, b2t)


def kernel(features, edge_index, W1, b1, W2, b2):
    ei = edge_index.astype(jnp.int32)
    rep = 128 // HID
    b1t = jnp.tile(b1, rep).reshape(1, 128)
    b2t = jnp.tile(b2, rep).reshape(1, 128)
    W2bd = jnp.kron(jnp.eye(rep, dtype=jnp.float32), W2)

    degp = _sc_degrees(ei)
    h1p, nrmd_p, nrms_p = _tc_mm1(features, W1, degp)
    p1 = _sc_agg(h1p.reshape(N_PAD, HID), ei)
    h2p = _tc_mid(p1.reshape(NC, PROWS, 128), nrmd_p, nrms_p, b1t, W2bd)
    p2 = _sc_agg(h2p.reshape(N_PAD, HID), ei)
    outp = _tc_final(p2.reshape(NC, PROWS, 128), nrmd_p, b2t)
    return outp.reshape(N_PAD, HID)[:N_NODES]


# speedup vs baseline: 20.5380x; 1.1956x over previous
import dataclasses
import functools

import jax
import jax.numpy as jnp
from jax import lax
from jax.experimental import pallas as pl
from jax.experimental.pallas import tpu as pltpu
from jax.experimental.pallas import tpu_sc as plsc

N_NODES = 10000
N_EDGES = 160000
F_IN = 256
HID = 16

NC, NS, LANES = 2, 16, 16
NW = NC * NS
WIN = 128
N_PAD = 10240
DUMP = N_NODES
ROWS_PER_TILE = N_PAD // NS
ZCHUNK = 128
NZ = ROWS_PER_TILE // ZCHUNK
EPT = N_EDGES // NW
VFULL = EPT // LANES
TAIL = EPT - VFULL * LANES

_mesh = plsc.VectorSubcoreMesh(core_axis_name="c", subcore_axis_name="s")

_sc_params = pltpu.CompilerParams(
    needs_layout_passes=False, use_tc_tiling_on_sc=False)


def _deg_body(ei, out, hs, hd, si, di):
    cid = lax.axis_index("c")
    sid = lax.axis_index("s")
    wid = cid * NS + sid
    zeros = jnp.zeros((LANES,), jnp.float32)

    @pl.loop(0, N_PAD // LANES)
    def _(i):
        hs[pl.ds(i * LANES, LANES)] = zeros
        hd[pl.ds(i * LANES, LANES)] = zeros

    pltpu.sync_copy(ei.at[0, pl.ds(wid * EPT, EPT)], si.at[pl.ds(0, EPT)])
    pltpu.sync_copy(ei.at[1, pl.ds(wid * EPT, EPT)], di.at[pl.ds(0, EPT)])
    ones = jnp.ones((LANES,), jnp.float32)

    @pl.loop(0, VFULL)
    def _(v):
        plsc.addupdate_scatter(hs, [si[pl.ds(v * LANES, LANES)]], ones)
        plsc.addupdate_scatter(hd, [di[pl.ds(v * LANES, LANES)]], ones)

    tmask = lax.iota(jnp.int32, LANES) < TAIL
    plsc.addupdate_scatter(hs, [si[pl.ds(VFULL * LANES, LANES)]], ones, mask=tmask)
    plsc.addupdate_scatter(hd, [di[pl.ds(VFULL * LANES, LANES)]], ones, mask=tmask)

    pltpu.sync_copy(hs, out.at[0, wid])
    pltpu.sync_copy(hd, out.at[1, wid])


def _sc_degrees(ei):
    k = pl.kernel(
        _deg_body,
        out_type=jax.ShapeDtypeStruct((2, NW, N_PAD), jnp.float32),
        mesh=_mesh,
        scratch_types=[
            pltpu.VMEM((N_PAD,), jnp.float32),
            pltpu.VMEM((N_PAD,), jnp.float32),
            pltpu.VMEM((VFULL * LANES + LANES,), jnp.int32),
            pltpu.VMEM((VFULL * LANES + LANES,), jnp.int32),
        ],
        compiler_params=_sc_params,
    )
    return k(ei)


def _agg_body(h, ei, out, si, di, rows, zb, acc, sem):
    cid = lax.axis_index("c")
    sid = lax.axis_index("s")
    wid = cid * NS + sid
    zeros = jnp.zeros((LANES,), jnp.float32)

    @pl.loop(0, ZCHUNK)
    def _(i):
        zb[i, :] = zeros

    @pl.loop(0, NZ)
    def _(kk):
        pltpu.sync_copy(zb, acc.at[pl.ds(sid * ROWS_PER_TILE + kk * ZCHUNK, ZCHUNK)])

    pltpu.sync_copy(ei.at[0, pl.ds(wid * EPT, EPT)], si)
    pltpu.sync_copy(ei.at[1, pl.ds(wid * EPT, EPT)], di)
    plsc.subcore_barrier()

    pltpu.async_copy(h.at[si], rows, sem).wait()
    pltpu.sync_copy(rows, acc.at[di], add=True)

    plsc.subcore_barrier()
    pltpu.sync_copy(acc.at[pl.ds(sid * ROWS_PER_TILE, ROWS_PER_TILE)],
                    out.at[cid, pl.ds(sid * ROWS_PER_TILE, ROWS_PER_TILE)])


def _sc_agg(h, ei):
    k = pl.kernel(
        _agg_body,
        out_type=jax.ShapeDtypeStruct((NC, N_PAD, HID), jnp.float32),
        mesh=_mesh,
        scratch_types=[
            pltpu.VMEM((EPT,), jnp.int32),
            pltpu.VMEM((EPT,), jnp.int32),
            pltpu.VMEM((EPT, HID), jnp.float32),
            pltpu.VMEM((ZCHUNK, HID), jnp.float32),
            pltpu.VMEM_SHARED((N_PAD, HID), jnp.float32),
            pltpu.SemaphoreType.DMA,
        ],
        compiler_params=_sc_params,
    )
    return k(h, ei)


_BM = 1024
_BMP = _BM // (128 // HID)
PROWS = N_PAD // (128 // HID)


def _packmask():
    r8 = 128 // HID
    m_row = lax.broadcasted_iota(jnp.int32, (_BM, 128), 0)
    c_col = lax.broadcasted_iota(jnp.int32, (_BM, 128), 1)
    return (m_row % r8 == c_col // HID).astype(jnp.float32)


def _grpsum(y):
    return jnp.sum(y.reshape(_BMP, 128 // HID, 128), axis=1)


def _mm1_body(x_ref, w1e_ref, degp_ref, hp_ref, nrmd_ref, nrms_ref):
    deg = jnp.maximum(jnp.sum(degp_ref[...], axis=1), 1.0)
    norm = lax.rsqrt(deg)
    mask = _packmask()
    nrms_col = norm[0][:, None]
    nrmd_col = norm[1][:, None]
    nrms_ref[...] = _grpsum(nrms_col * mask)
    nrmd_ref[...] = _grpsum(nrmd_col * mask)
    mm = jnp.dot(x_ref[...], w1e_ref[...], preferred_element_type=jnp.float32)
    hp_ref[...] = _grpsum(mm * (nrms_col * mask))


def _tc_mm1(features, W1e, degp):
    grid = N_PAD // _BM
    return pl.pallas_call(
        _mm1_body,
        grid=(grid,),
        in_specs=[
            pl.BlockSpec((_BM, F_IN), lambda i: (i, 0)),
            pl.BlockSpec((F_IN, 128), lambda i: (0, 0)),
            pl.BlockSpec((2, NW, _BM), lambda i: (0, 0, i)),
        ],
        out_specs=[
            pl.BlockSpec((_BMP, 128), lambda i: (i, 0)),
            pl.BlockSpec((_BMP, 128), lambda i: (i, 0)),
            pl.BlockSpec((_BMP, 128), lambda i: (i, 0)),
        ],
        out_shape=[
            jax.ShapeDtypeStruct((PROWS, 128), jnp.float32),
            jax.ShapeDtypeStruct((PROWS, 128), jnp.float32),
            jax.ShapeDtypeStruct((PROWS, 128), jnp.float32),
        ],
    )(features, W1e, degp)


def _mid_body(p_ref, nrmd_ref, nrms_ref, b1_ref, w2bd_ref, h2p_ref):
    agg = p_ref[0] + p_ref[1]
    t = jnp.maximum(agg * nrmd_ref[...] + b1_ref[...], 0.0)
    mm = jnp.dot(t, w2bd_ref[...], preferred_element_type=jnp.float32)
    h2p_ref[...] = mm * nrms_ref[...]


def _tc_mid(p1, nrmd_p, nrms_p, b1t, W2bd):
    grid = PROWS // _BMP
    return pl.pallas_call(
        _mid_body,
        grid=(grid,),
        in_specs=[
            pl.BlockSpec((NC, _BMP, 128), lambda i: (0, i, 0)),
            pl.BlockSpec((_BMP, 128), lambda i: (i, 0)),
            pl.BlockSpec((_BMP, 128), lambda i: (i, 0)),
            pl.BlockSpec((1, 128), lambda i: (0, 0)),
            pl.BlockSpec((128, 128), lambda i: (0, 0)),
        ],
        out_specs=pl.BlockSpec((_BMP, 128), lambda i: (i, 0)),
        out_shape=jax.ShapeDtypeStruct((PROWS, 128), jnp.float32),
    )(p1, nrmd_p, nrms_p, b1t, W2bd)


def _final_body(p_ref, nrmd_ref, b2_ref, out_ref):
    agg = p_ref[0] + p_ref[1]
    out_ref[...] = agg * nrmd_ref[...] + b2_ref[...]


def _tc_final(p2, nrmd_p, b2t):
    grid = PROWS // _BMP
    return pl.pallas_call(
        _final_body,
        grid=(grid,),
        in_specs=[
            pl.BlockSpec((NC, _BMP, 128), lambda i: (0, i, 0)),
            pl.BlockSpec((_BMP, 128), lambda i: (i, 0)),
            pl.BlockSpec((1, 128), lambda i: (0, 0)),
        ],
        out_specs=pl.BlockSpec((_BMP, 128), lambda i: (i, 0)),
        out_shape=jax.ShapeDtypeStruct((PROWS, 128), jnp.float32),
    )(p2, nrmd_p, b2t)


def kernel(features, edge_index, W1, b1, W2, b2):
    ei = edge_index.astype(jnp.int32)
    rep = 128 // HID
    b1t = jnp.tile(b1, rep).reshape(1, 128)
    b2t = jnp.tile(b2, rep).reshape(1, 128)
    W1e = jnp.tile(W1, (1, rep))
    W2bd = jnp.kron(jnp.eye(rep, dtype=jnp.float32), W2)

    degp = _sc_degrees(ei)
    h1p, nrmd_p, nrms_p = _tc_mm1(features, W1e, degp)
    p1 = _sc_agg(h1p.reshape(N_PAD, HID), ei)
    h2p = _tc_mid(p1.reshape(NC, PROWS, 128), nrmd_p, nrms_p, b1t, W2bd)
    p2 = _sc_agg(h2p.reshape(N_PAD, HID), ei)
    outp = _tc_final(p2.reshape(NC, PROWS, 128), nrmd_p, b2t)
    return outp.reshape(N_PAD, HID)[:N_NODES]
